# baseline (speedup 1.0000x reference)
import jax
import jax.numpy as jnp
from jax.experimental import pallas as pl
from jax.experimental.pallas import tpu as pltpu

H = 16
DH = 128
BQ = 2048
LOG2E = 1.4426950408889634


def _proj_kernel(x_ref, w_ref, o_ref, xb_ref):
    @pl.when(pl.program_id(0) == 0)
    def _():
        xb_ref[...] = x_ref[...].astype(jnp.bfloat16)

    w = w_ref[...].astype(jnp.bfloat16)
    o_ref[...] = jax.lax.dot_general(
        xb_ref[...], w, (((1,), (1,)), ((), ())),
        preferred_element_type=jnp.float32).astype(jnp.bfloat16)


def _attn_kernel(q_ref, k_ref, v_ref, o_ref):
    c = LOG2E / (DH ** 0.5)
    for p in range(4):
        sl = slice(p * DH, (p + 1) * DH)
        s = jax.lax.dot_general(
            q_ref[:, sl], k_ref[:, sl], (((1,), (1,)), ((), ())),
            preferred_element_type=jnp.float32)
        e = jnp.exp2(s * c)
        denom = jnp.sum(e, axis=1, keepdims=True)
        o_h = jax.lax.dot_general(
            e.astype(jnp.bfloat16), v_ref[:, sl], (((1,), (0,)), ((), ())),
            preferred_element_type=jnp.float32)
        o_ref[:, sl] = (o_h * (1.0 / denom)).astype(jnp.bfloat16)


def _oproj_kernel(x_ref, w_ref, b_ref, o_ref):
    w = w_ref[...].astype(jnp.bfloat16)
    o_ref[...] = jax.lax.dot_general(
        x_ref[...], w, (((1,), (1,)), ((), ())),
        preferred_element_type=jnp.float32) + b_ref[...]


def _project(x, w, bn=512):
    l, d = x.shape
    n = w.shape[0]
    return pl.pallas_call(
        _proj_kernel,
        grid=(n // bn,),
        in_specs=[
            pl.BlockSpec((l, d), lambda j: (0, 0)),
            pl.BlockSpec((bn, d), lambda j: (j, 0)),
        ],
        out_specs=pl.BlockSpec((l, bn), lambda j: (0, j)),
        out_shape=jax.ShapeDtypeStruct((l, n), jnp.bfloat16),
        scratch_shapes=[pltpu.VMEM((l, d), jnp.bfloat16)],
    )(x, w)


def kernel(query, key, value, Wq, Wk, Wv, Wo, bo):
    b, l, d = query.shape

    q = _project(query[0], Wq)
    k = _project(key[0], Wk)
    v = _project(value[0], Wv)

    attn = pl.pallas_call(
        _attn_kernel,
        grid=(H // 4, l // BQ),
        in_specs=[
            pl.BlockSpec((BQ, 4 * DH), lambda h, i: (i, h)),
            pl.BlockSpec((l, 4 * DH), lambda h, i: (0, h)),
            pl.BlockSpec((l, 4 * DH), lambda h, i: (0, h)),
        ],
        out_specs=pl.BlockSpec((BQ, 4 * DH), lambda h, i: (i, h)),
        out_shape=jax.ShapeDtypeStruct((l, d), jnp.bfloat16),
    )(q, k, v)

    bn = 512
    out = pl.pallas_call(
        _oproj_kernel,
        grid=(d // bn,),
        in_specs=[
            pl.BlockSpec((l, d), lambda j: (0, 0)),
            pl.BlockSpec((bn, d), lambda j: (j, 0)),
            pl.BlockSpec((1, bn), lambda j: (0, j)),
        ],
        out_specs=pl.BlockSpec((l, bn), lambda j: (0, j)),
        out_shape=jax.ShapeDtypeStruct((l, d), jnp.float32),
    )(attn, Wo, bo.reshape(1, d))

    return out.reshape(b, l, d)


# speedup vs baseline: 1.0467x; 1.0467x over previous
import jax
import jax.numpy as jnp
from jax.experimental import pallas as pl
from jax.experimental.pallas import tpu as pltpu

H = 16
DH = 128
BQ = 2048
LOG2E = 1.4426950408889634


def _proj_kernel(x_ref, w_ref, o_ref, xb_ref):
    @pl.when(pl.program_id(0) == 0)
    def _():
        xb_ref[...] = x_ref[...].astype(jnp.bfloat16)

    w = w_ref[...].astype(jnp.bfloat16)
    o_ref[...] = jax.lax.dot_general(
        xb_ref[...], w, (((1,), (1,)), ((), ())),
        preferred_element_type=jnp.float32).astype(jnp.bfloat16)


def _attn_kernel(q_ref, k_ref, v_ref, o_ref):
    c = LOG2E / (DH ** 0.5)
    for p in range(2):
        sl = slice(p * DH, (p + 1) * DH)
        s = jax.lax.dot_general(
            q_ref[:, sl], k_ref[:, sl], (((1,), (1,)), ((), ())),
            preferred_element_type=jnp.float32)
        e = jnp.exp2(s * c)
        denom = jnp.sum(e, axis=1, keepdims=True)
        o_h = jax.lax.dot_general(
            e.astype(jnp.bfloat16), v_ref[:, sl], (((1,), (0,)), ((), ())),
            preferred_element_type=jnp.float32)
        o_ref[:, sl] = (o_h * (1.0 / denom)).astype(jnp.bfloat16)


def _oproj_kernel(x_ref, w_ref, b_ref, o_ref):
    w = w_ref[...].astype(jnp.bfloat16)
    o_ref[...] = jax.lax.dot_general(
        x_ref[...], w, (((1,), (1,)), ((), ())),
        preferred_element_type=jnp.float32) + b_ref[...]


def _project(x, w, bn=512):
    l, d = x.shape
    n = w.shape[0]
    return pl.pallas_call(
        _proj_kernel,
        grid=(n // bn,),
        in_specs=[
            pl.BlockSpec((l, d), lambda j: (0, 0)),
            pl.BlockSpec((bn, d), lambda j: (j, 0)),
        ],
        out_specs=pl.BlockSpec((l, bn), lambda j: (0, j)),
        out_shape=jax.ShapeDtypeStruct((l, n), jnp.bfloat16),
        scratch_shapes=[pltpu.VMEM((l, d), jnp.bfloat16)],
    )(x, w)


def kernel(query, key, value, Wq, Wk, Wv, Wo, bo):
    b, l, d = query.shape

    q = _project(query[0], Wq)
    k = _project(key[0], Wk)
    v = _project(value[0], Wv)

    attn = pl.pallas_call(
        _attn_kernel,
        grid=(H // 2, l // BQ),
        in_specs=[
            pl.BlockSpec((BQ, 2 * DH), lambda h, i: (i, h)),
            pl.BlockSpec((l, 2 * DH), lambda h, i: (0, h)),
            pl.BlockSpec((l, 2 * DH), lambda h, i: (0, h)),
        ],
        out_specs=pl.BlockSpec((BQ, 2 * DH), lambda h, i: (i, h)),
        out_shape=jax.ShapeDtypeStruct((l, d), jnp.bfloat16),
    )(q, k, v)

    bn = 512
    out = pl.pallas_call(
        _oproj_kernel,
        grid=(d // bn,),
        in_specs=[
            pl.BlockSpec((l, d), lambda j: (0, 0)),
            pl.BlockSpec((bn, d), lambda j: (j, 0)),
            pl.BlockSpec((1, bn), lambda j: (0, j)),
        ],
        out_specs=pl.BlockSpec((l, bn), lambda j: (0, j)),
        out_shape=jax.ShapeDtypeStruct((l, d), jnp.float32),
    )(attn, Wo, bo.reshape(1, d))

    return out.reshape(b, l, d)


# speedup vs baseline: 1.0491x; 1.0024x over previous
import jax
import jax.numpy as jnp
from jax.experimental import pallas as pl
from jax.experimental.pallas import tpu as pltpu

H = 16
DH = 128
BQ = 2048
LOG2E = 1.4426950408889634


def _proj_kernel(x_ref, w_ref, o_ref, xb_ref):
    @pl.when(pl.program_id(0) == 0)
    def _():
        xb_ref[...] = x_ref[...].astype(jnp.bfloat16)

    w = w_ref[...].astype(jnp.bfloat16)
    o_ref[...] = jax.lax.dot_general(
        xb_ref[...], w, (((1,), (1,)), ((), ())),
        preferred_element_type=jnp.float32).astype(jnp.bfloat16)


def _attn_kernel(q_ref, k_ref, v_ref, o_ref):
    c = LOG2E / (DH ** 0.5)
    for p in range(2):
        sl = slice(p * DH, (p + 1) * DH)
        s = jax.lax.dot_general(
            q_ref[:, sl], k_ref[:, sl], (((1,), (1,)), ((), ())),
            preferred_element_type=jnp.float32)
        e = jnp.exp2(s * c)
        denom = jnp.sum(e, axis=1, keepdims=True)
        o_h = jax.lax.dot_general(
            e.astype(jnp.bfloat16), v_ref[:, sl], (((1,), (0,)), ((), ())),
            preferred_element_type=jnp.float32)
        o_ref[:, sl] = (o_h * (1.0 / denom)).astype(jnp.bfloat16)


def _oproj_kernel(x_ref, w_ref, b_ref, o_ref):
    w = w_ref[...].astype(jnp.bfloat16)
    o_ref[...] = jax.lax.dot_general(
        x_ref[...], w, (((1,), (1,)), ((), ())),
        preferred_element_type=jnp.float32) + b_ref[...]


def _project(x, w, bn=512):
    l, d = x.shape
    n = w.shape[0]
    return pl.pallas_call(
        _proj_kernel,
        grid=(n // bn,),
        in_specs=[
            pl.BlockSpec((l, d), lambda j: (0, 0)),
            pl.BlockSpec((bn, d), lambda j: (j, 0)),
        ],
        out_specs=pl.BlockSpec((l, bn), lambda j: (0, j)),
        out_shape=jax.ShapeDtypeStruct((l, n), jnp.bfloat16),
        scratch_shapes=[pltpu.VMEM((l, d), jnp.bfloat16)],
    )(x, w)


def kernel(query, key, value, Wq, Wk, Wv, Wo, bo):
    b, l, d = query.shape

    q = _project(query[0], Wq)
    k = _project(key[0], Wk)
    v = _project(value[0], Wv)

    attn = pl.pallas_call(
        _attn_kernel,
        grid=(H // 2, l // BQ),
        in_specs=[
            pl.BlockSpec((BQ, 2 * DH), lambda h, i: (i, h)),
            pl.BlockSpec((l, 2 * DH), lambda h, i: (0, h)),
            pl.BlockSpec((l, 2 * DH), lambda h, i: (0, h)),
        ],
        out_specs=pl.BlockSpec((BQ, 2 * DH), lambda h, i: (i, h)),
        out_shape=jax.ShapeDtypeStruct((l, d), jnp.bfloat16),
    )(q, k, v)

    bn = 256
    out = pl.pallas_call(
        _oproj_kernel,
        grid=(d // bn,),
        in_specs=[
            pl.BlockSpec((l, d), lambda j: (0, 0)),
            pl.BlockSpec((bn, d), lambda j: (j, 0)),
            pl.BlockSpec((1, bn), lambda j: (0, j)),
        ],
        out_specs=pl.BlockSpec((l, bn), lambda j: (0, j)),
        out_shape=jax.ShapeDtypeStruct((l, d), jnp.float32),
    )(attn, Wo, bo.reshape(1, d))

    return out.reshape(b, l, d)
